# baseline (speedup 1.0000x reference)
import functools

import jax
import jax.numpy as jnp
from jax import lax
from jax.experimental import pallas as pl
from jax.experimental.pallas import tpu as pltpu
from jax.experimental.pallas import tpu_sc as plsc

BATCH = 16384
EMB = 64
LANES = 16

_info = plsc.get_sparse_core_info()
NC = _info.num_cores
NS = _info.num_subcores
NW = NC * NS
BPW = BATCH // NW
CHUNK = 128
NCHUNK = BPW // CHUNK
NBLK = BPW // LANES


def _body(u_hbm, v_hbm, uemb_hbm, vemb_hbm, out_hbm,
          uidx, vidx, urows, vrows, outv, sem):
    wid = lax.axis_index("s") * NC + lax.axis_index("c")
    crow = wid * NCHUNK

    for j in range(NCHUNK):
        pltpu.sync_copy(u_hbm.at[crow + j], uidx.at[j])
        pltpu.sync_copy(v_hbm.at[crow + j], vidx.at[j])

    copies = []
    for j in range(NCHUNK):
        copies.append(pltpu.async_copy(
            uemb_hbm.at[uidx.at[j]], urows.at[pl.ds(j * CHUNK, CHUNK)], sem))
        copies.append(pltpu.async_copy(
            vemb_hbm.at[vidx.at[j]], vrows.at[pl.ds(j * CHUNK, CHUNK)], sem))
    for c in copies:
        c.wait()

    iota = lax.iota(jnp.int32, LANES)

    def block(r, _):
        ridx = iota + r * LANES
        acc = [jnp.zeros((LANES,), jnp.float32) for _ in range(4)]
        for j in range(EMB):
            col = jnp.full((LANES,), j, jnp.int32)
            uc = plsc.load_gather(urows, [ridx, col])
            vc = plsc.load_gather(vrows, [ridx, col])
            acc[j % 4] = acc[j % 4] + uc * vc
        outv[pl.ds(r * LANES, LANES)] = (acc[0] + acc[1]) + (acc[2] + acc[3])
        return 0

    lax.fori_loop(0, NBLK, block, 0)

    pltpu.sync_copy(outv, out_hbm.at[pl.ds(wid * BPW, BPW)])


@functools.partial(
    pl.kernel,
    out_type=jax.ShapeDtypeStruct((BATCH,), jnp.float32),
    mesh=plsc.VectorSubcoreMesh(core_axis_name="c", subcore_axis_name="s"),
    compiler_params=pltpu.CompilerParams(
        needs_layout_passes=False, use_tc_tiling_on_sc=False),
    scratch_types=[
        pltpu.VMEM((NCHUNK, CHUNK), jnp.int32),
        pltpu.VMEM((NCHUNK, CHUNK), jnp.int32),
        pltpu.VMEM((BPW, EMB), jnp.float32),
        pltpu.VMEM((BPW, EMB), jnp.float32),
        pltpu.VMEM((BPW,), jnp.float32),
        pltpu.SemaphoreType.DMA,
    ],
)
def _mf_kernel(u_hbm, v_hbm, uemb_hbm, vemb_hbm, out_hbm,
               uidx, vidx, urows, vrows, outv, sem):
    _body(u_hbm, v_hbm, uemb_hbm, vemb_hbm, out_hbm,
          uidx, vidx, urows, vrows, outv, sem)


def kernel(u, v, user_emb, item_emb):
    u2 = u.astype(jnp.int32).reshape(BATCH // CHUNK, CHUNK)
    v2 = v.astype(jnp.int32).reshape(BATCH // CHUNK, CHUNK)
    return _mf_kernel(u2, v2, user_emb, item_emb)


# speedup vs baseline: 6.9175x; 6.9175x over previous
import functools

import jax
import jax.numpy as jnp
from jax import lax
from jax.experimental import pallas as pl
from jax.experimental.pallas import tpu as pltpu
from jax.experimental.pallas import tpu_sc as plsc

BATCH = 16384
NROWS = 1000000
EMB = 64
LANES = 16

_info = plsc.get_sparse_core_info()
NC = _info.num_cores
NS = _info.num_subcores
NW = NC * NS
BPW = BATCH // NW
NBLK = BPW // LANES


def _body(u_hbm, v_hbm, ut_hbm, vt_hbm, out_hbm,
          uidx, vidx, wu_a, wv_a, wu_b, wv_b, outv, sem_a, sem_b):
    wid = lax.axis_index("s") * NC + lax.axis_index("c")
    crow = wid * (BPW // 128)

    for j in range(BPW // 128):
        pltpu.sync_copy(u_hbm.at[crow + j], uidx.at[j])
        pltpu.sync_copy(v_hbm.at[crow + j], vidx.at[j])

    iota = lax.iota(jnp.int32, LANES)

    def idx_vec(ref, b):
        return ref[b // 8, pl.ds((b % 8) * LANES, LANES)]

    def stage(b, wu, wv, sem):
        uvec = idx_vec(uidx, b)
        vvec = idx_vec(vidx, b)
        for tab, vec, w in ((ut_hbm, uvec, wu), (vt_hbm, vvec, wv)):
            for k in range(LANES):
                i_k = vec[k]
                o128 = pl.multiple_of(i_k & -128, 128)
                o16 = ((i_k >> 4) & 7) * LANES
                src = tab.at[:, :, pl.ds(o128, 128)].at[:, :, pl.ds(o16, LANES)]
                dst = w.at[k // 8].at[:, :, pl.ds((k % 8) * LANES, LANES)]
                pltpu.async_copy(src, dst, sem)

    def drain(wu, wv, sem):
        dummy = ut_hbm.at[:, :, pl.ds(0, 128)].at[:, :, pl.ds(0, LANES)]
        for w in (wu, wv):
            for k in range(LANES):
                dst = w.at[k // 8].at[:, :, pl.ds((k % 8) * LANES, LANES)]
                pltpu.make_async_copy(dummy, dst, sem).wait()

    def compute(b, wu, wv):
        uvec = idx_vec(uidx, b)
        vvec = idx_vec(vidx, b)
        kk = iota >> 3
        slot = (iota & 7) * LANES
        offs_u = slot + (uvec & (LANES - 1))
        offs_v = slot + (vvec & (LANES - 1))
        acc = [jnp.zeros((LANES,), jnp.float32) for _ in range(4)]
        for j in range(EMB):
            tj = jnp.full((LANES,), j // 8, jnp.int32)
            r = jnp.full((LANES,), j % 8, jnp.int32)
            uc = plsc.load_gather(wu, [kk, tj, r, offs_u])
            vc = plsc.load_gather(wv, [kk, tj, r, offs_v])
            acc[j % 4] = acc[j % 4] + uc * vc
        outv[pl.ds(b * LANES, LANES)] = (acc[0] + acc[1]) + (acc[2] + acc[3])

    stage(0, wu_a, wv_a, sem_a)

    def pipelined(i, _):
        stage(2 * i + 1, wu_b, wv_b, sem_b)
        drain(wu_a, wv_a, sem_a)
        compute(2 * i, wu_a, wv_a)

        @pl.when(i < NBLK // 2 - 1)
        def _():
            stage(2 * i + 2, wu_a, wv_a, sem_a)

        drain(wu_b, wv_b, sem_b)
        compute(2 * i + 1, wu_b, wv_b)
        return 0

    lax.fori_loop(0, NBLK // 2, pipelined, 0)

    pltpu.sync_copy(outv, out_hbm.at[pl.ds(wid * BPW, BPW)])


@functools.partial(
    pl.kernel,
    out_type=jax.ShapeDtypeStruct((BATCH,), jnp.float32),
    mesh=plsc.VectorSubcoreMesh(core_axis_name="c", subcore_axis_name="s"),
    compiler_params=pltpu.CompilerParams(needs_layout_passes=False),
    scratch_types=[
        pltpu.VMEM((BPW // 128, 128), jnp.int32),
        pltpu.VMEM((BPW // 128, 128), jnp.int32),
        pltpu.VMEM((2, 8, 8, 128), jnp.float32),
        pltpu.VMEM((2, 8, 8, 128), jnp.float32),
        pltpu.VMEM((2, 8, 8, 128), jnp.float32),
        pltpu.VMEM((2, 8, 8, 128), jnp.float32),
        pltpu.VMEM((BPW,), jnp.float32),
        pltpu.SemaphoreType.DMA,
        pltpu.SemaphoreType.DMA,
    ],
)
def _mf_kernel(u_hbm, v_hbm, ut_hbm, vt_hbm, out_hbm,
               uidx, vidx, wu_a, wv_a, wu_b, wv_b, outv, sem_a, sem_b):
    _body(u_hbm, v_hbm, ut_hbm, vt_hbm, out_hbm,
          uidx, vidx, wu_a, wv_a, wu_b, wv_b, outv, sem_a, sem_b)


def kernel(u, v, user_emb, item_emb):
    u2 = u.astype(jnp.int32).reshape(BATCH // 128, 128)
    v2 = v.astype(jnp.int32).reshape(BATCH // 128, 128)
    ut = user_emb.T.reshape(8, 8, NROWS)
    vt = item_emb.T.reshape(8, 8, NROWS)
    return _mf_kernel(u2, v2, ut, vt)


# speedup vs baseline: 7.1845x; 1.0386x over previous
import functools

import jax
import jax.numpy as jnp
from jax import lax
from jax.experimental import pallas as pl
from jax.experimental.pallas import tpu as pltpu
from jax.experimental.pallas import tpu_sc as plsc

BATCH = 16384
NROWS = 1000000
EMB = 64
LANES = 16

_info = plsc.get_sparse_core_info()
NC = _info.num_cores
NS = _info.num_subcores
NW = NC * NS
BPW = BATCH // NW
NBLK = BPW // LANES


def _body(u_hbm, v_hbm, ut_hbm, vt_hbm, out_hbm,
          uidx, vidx, wu_a, wv_a, wu_b, wv_b, outv, sem_a, sem_b):
    wid = lax.axis_index("s") * NC + lax.axis_index("c")
    crow = wid * (BPW // 128)

    for j in range(BPW // 128):
        pltpu.sync_copy(u_hbm.at[crow + j], uidx.at[j])
        pltpu.sync_copy(v_hbm.at[crow + j], vidx.at[j])

    iota = lax.iota(jnp.int32, LANES)

    def idx_vec(ref, b):
        return ref[b // 8, pl.ds((b % 8) * LANES, LANES)]

    def stage(b, wu, wv, sem):
        uvec = idx_vec(uidx, b)
        vvec = idx_vec(vidx, b)
        for vec, tab, w in ((uvec, ut_hbm, wu), (vvec, vt_hbm, wv)):
            for k in range(LANES):
                i_k = vec[k]
                o128 = pl.multiple_of(i_k & -128, 128)
                o16 = ((i_k >> 4) & 7) * LANES
                src = tab.at[:, :, pl.ds(o128, 128)].at[:, :, pl.ds(o16, LANES)]
                pltpu.async_copy(
                    src, w.at[:, :, pl.ds(k * LANES, LANES)], sem)

    def drain(wu, wv, sem):
        dummy = ut_hbm.at[:, :, pl.ds(0, 256)]
        pltpu.make_async_copy(dummy, wu, sem).wait()
        pltpu.make_async_copy(dummy, wv, sem).wait()

    def compute(b, wu, wv):
        offs_u = iota * LANES + (idx_vec(uidx, b) & (LANES - 1))
        offs_v = iota * LANES + (idx_vec(vidx, b) & (LANES - 1))
        acc = [jnp.zeros((LANES,), jnp.float32) for _ in range(4)]
        for j in range(EMB):
            tj = jnp.full((LANES,), j // 8, jnp.int32)
            r = jnp.full((LANES,), j % 8, jnp.int32)
            uc = plsc.load_gather(wu, [tj, r, offs_u])
            vc = plsc.load_gather(wv, [tj, r, offs_v])
            acc[j % 4] = acc[j % 4] + uc * vc
        outv[pl.ds(b * LANES, LANES)] = (acc[0] + acc[1]) + (acc[2] + acc[3])

    stage(0, wu_a, wv_a, sem_a)

    def pipelined(i, _):
        stage(2 * i + 1, wu_b, wv_b, sem_b)
        drain(wu_a, wv_a, sem_a)
        compute(2 * i, wu_a, wv_a)

        @pl.when(i < NBLK // 2 - 1)
        def _():
            stage(2 * i + 2, wu_a, wv_a, sem_a)

        drain(wu_b, wv_b, sem_b)
        compute(2 * i + 1, wu_b, wv_b)
        return 0

    lax.fori_loop(0, NBLK // 2, pipelined, 0)

    pltpu.sync_copy(outv, out_hbm.at[pl.ds(wid * BPW, BPW)])


@functools.partial(
    pl.kernel,
    out_type=jax.ShapeDtypeStruct((BATCH,), jnp.float32),
    mesh=plsc.VectorSubcoreMesh(core_axis_name="c", subcore_axis_name="s"),
    compiler_params=pltpu.CompilerParams(needs_layout_passes=False),
    scratch_types=[
        pltpu.VMEM((BPW // 128, 128), jnp.int32),
        pltpu.VMEM((BPW // 128, 128), jnp.int32),
        pltpu.VMEM((8, 8, 16 * LANES), jnp.float32),
        pltpu.VMEM((8, 8, 16 * LANES), jnp.float32),
        pltpu.VMEM((8, 8, 16 * LANES), jnp.float32),
        pltpu.VMEM((8, 8, 16 * LANES), jnp.float32),
        pltpu.VMEM((BPW,), jnp.float32),
        pltpu.SemaphoreType.DMA,
        pltpu.SemaphoreType.DMA,
    ],
)
def _mf_kernel(u_hbm, v_hbm, ut_hbm, vt_hbm, out_hbm,
               uidx, vidx, wu_a, wv_a, wu_b, wv_b, outv, sem_a, sem_b):
    _body(u_hbm, v_hbm, ut_hbm, vt_hbm, out_hbm,
          uidx, vidx, wu_a, wv_a, wu_b, wv_b, outv, sem_a, sem_b)


def kernel(u, v, user_emb, item_emb):
    u2 = u.astype(jnp.int32).reshape(BATCH // 128, 128)
    v2 = v.astype(jnp.int32).reshape(BATCH // 128, 128)
    ut = user_emb.T.reshape(8, 8, NROWS)
    vt = item_emb.T.reshape(8, 8, NROWS)
    return _mf_kernel(u2, v2, ut, vt)


# speedup vs baseline: 7.3997x; 1.0300x over previous
import functools

import jax
import jax.numpy as jnp
from jax import lax
from jax.experimental import pallas as pl
from jax.experimental.pallas import tpu as pltpu
from jax.experimental.pallas import tpu_sc as plsc

BATCH = 16384
NROWS = 1000000
EMB = 64
LANES = 16
W = 8
VPB = 1
BLANES = VPB * LANES * W

_info = plsc.get_sparse_core_info()
NC = _info.num_cores
NS = _info.num_subcores
NW = NC * NS
BPW = BATCH // NW
NVEC = BPW // LANES
NRND = NVEC // VPB


def _body(u_hbm, v_hbm, ut_hbm, vt_hbm, out_hbm,
          uidx, vidx, wu_a, wv_a, wu_b, wv_b, outv, sem_a, sem_b):
    wid = lax.axis_index("s") * NC + lax.axis_index("c")
    crow = wid * (BPW // 128)

    for j in range(BPW // 128):
        pltpu.sync_copy(u_hbm.at[crow + j], uidx.at[j])
        pltpu.sync_copy(v_hbm.at[crow + j], vidx.at[j])

    iota = lax.iota(jnp.int32, LANES)

    def idx_vec(ref, t):
        return ref[t // 8, pl.ds((t % 8) * LANES, LANES)]

    def stage(r, wu, wv, sem):
        for t in range(VPB):
            uvec = idx_vec(uidx, r * VPB + t)
            vvec = idx_vec(vidx, r * VPB + t)
            for vec, tab, w in ((uvec, ut_hbm, wu), (vvec, vt_hbm, wv)):
                for k in range(LANES):
                    i_k = vec[k]
                    o128 = pl.multiple_of(i_k & -128, 128)
                    o8 = (i_k >> 3 & 15) * W
                    src = tab.at[:, :, pl.ds(o128, 128)].at[:, :, pl.ds(o8, W)]
                    dst = w.at[:, :, pl.ds((t * LANES + k) * W, W)]
                    pltpu.async_copy(src, dst, sem)

    def drain(wu, wv, sem):
        dummy = ut_hbm.at[:, :, pl.ds(0, BLANES)]
        pltpu.make_async_copy(dummy, wu, sem).wait()
        pltpu.make_async_copy(dummy, wv, sem).wait()

    def compute(r, wu, wv):
        for t in range(VPB):
            base = iota * W + t * (LANES * W)
            offs_u = base + (idx_vec(uidx, r * VPB + t) & (W - 1))
            offs_v = base + (idx_vec(vidx, r * VPB + t) & (W - 1))
            acc = [jnp.zeros((LANES,), jnp.float32) for _ in range(4)]
            for j in range(EMB):
                tj = jnp.full((LANES,), j // 8, jnp.int32)
                rw = jnp.full((LANES,), j % 8, jnp.int32)
                uc = plsc.load_gather(wu, [tj, rw, offs_u])
                vc = plsc.load_gather(wv, [tj, rw, offs_v])
                acc[j % 4] = acc[j % 4] + uc * vc
            outv[pl.ds((r * VPB + t) * LANES, LANES)] = (
                (acc[0] + acc[1]) + (acc[2] + acc[3]))

    stage(0, wu_a, wv_a, sem_a)

    def pipelined(i, _):
        stage(2 * i + 1, wu_b, wv_b, sem_b)
        drain(wu_a, wv_a, sem_a)
        compute(2 * i, wu_a, wv_a)

        @pl.when(i < NRND // 2 - 1)
        def _():
            stage(2 * i + 2, wu_a, wv_a, sem_a)

        drain(wu_b, wv_b, sem_b)
        compute(2 * i + 1, wu_b, wv_b)
        return 0

    lax.fori_loop(0, NRND // 2, pipelined, 0)

    pltpu.sync_copy(outv, out_hbm.at[pl.ds(wid * BPW, BPW)])


@functools.partial(
    pl.kernel,
    out_type=jax.ShapeDtypeStruct((BATCH,), jnp.float32),
    mesh=plsc.VectorSubcoreMesh(core_axis_name="c", subcore_axis_name="s"),
    compiler_params=pltpu.CompilerParams(needs_layout_passes=False),
    scratch_types=[
        pltpu.VMEM((BPW // 128, 128), jnp.int32),
        pltpu.VMEM((BPW // 128, 128), jnp.int32),
        pltpu.VMEM((8, 8, BLANES), jnp.float32),
        pltpu.VMEM((8, 8, BLANES), jnp.float32),
        pltpu.VMEM((8, 8, BLANES), jnp.float32),
        pltpu.VMEM((8, 8, BLANES), jnp.float32),
        pltpu.VMEM((BPW,), jnp.float32),
        pltpu.SemaphoreType.DMA,
        pltpu.SemaphoreType.DMA,
    ],
)
def _mf_kernel(u_hbm, v_hbm, ut_hbm, vt_hbm, out_hbm,
               uidx, vidx, wu_a, wv_a, wu_b, wv_b, outv, sem_a, sem_b):
    _body(u_hbm, v_hbm, ut_hbm, vt_hbm, out_hbm,
          uidx, vidx, wu_a, wv_a, wu_b, wv_b, outv, sem_a, sem_b)


def kernel(u, v, user_emb, item_emb):
    u2 = u.astype(jnp.int32).reshape(BATCH // 128, 128)
    v2 = v.astype(jnp.int32).reshape(BATCH // 128, 128)
    ut = user_emb.T.reshape(8, 8, NROWS)
    vt = item_emb.T.reshape(8, 8, NROWS)
    return _mf_kernel(u2, v2, ut, vt)
